# baseline (speedup 1.0000x reference)
import functools

import jax
import jax.numpy as jnp
from jax import lax
from jax.experimental import pallas as pl
from jax.experimental.pallas import tpu as pltpu
from jax.experimental.pallas import tpu_sc as plsc

D = 32
E = 160000
N = 10000
G = 64
NC, NS = 2, 16
NW = NC * NS
BPW = E // NW
CH = 1000
NCH = BPW // CH
RPT = N // NS

@functools.lru_cache(maxsize=None)
def _mesh():
    return plsc.VectorSubcoreMesh(core_axis_name="c", subcore_axis_name="s",
                                  num_cores=NC, num_subcores=NS)


@functools.lru_cache(maxsize=None)
def _gather_kernel():
    @functools.partial(
        pl.kernel,
        out_type=jax.ShapeDtypeStruct((E, D), jnp.float32),
        mesh=_mesh(),
        scratch_types=[
            pltpu.VMEM((CH,), jnp.int32),
            pltpu.VMEM((CH, D), jnp.float32),
            pltpu.SemaphoreType.DMA,
        ],
        compiler_params=pltpu.CompilerParams(use_tc_tiling_on_sc=False),
    )
    def _gather(table_hbm, idx_hbm, out_hbm, idx_v, rows_v, sem):
        wid = lax.axis_index("s") * NC + lax.axis_index("c")
        for j in range(NCH):
            b = wid * BPW + j * CH
            pltpu.sync_copy(idx_hbm.at[pl.ds(b, CH)], idx_v)
            pltpu.async_copy(table_hbm.at[idx_v], rows_v, sem).wait()
            pltpu.sync_copy(rows_v, out_hbm.at[pl.ds(b, CH)])

    return _gather


@functools.lru_cache(maxsize=None)
def _scatter_kernel(w):
    @functools.partial(
        pl.kernel,
        out_type=jax.ShapeDtypeStruct((NC, N, w), jnp.float32),
        mesh=_mesh(),
        scratch_types=[
            pltpu.VMEM((CH,), jnp.int32),
            pltpu.VMEM((CH, w), jnp.float32),
            pltpu.VMEM_SHARED((N, w), jnp.float32),
        ],
        compiler_params=pltpu.CompilerParams(use_tc_tiling_on_sc=False),
    )
    def _scatter(msg_hbm, idx_hbm, zero_hbm, out_hbm, idx_v, msg_v, acc):
        cid = lax.axis_index("c")
        sid = lax.axis_index("s")
        r0 = sid * RPT
        pltpu.sync_copy(zero_hbm.at[pl.ds(r0, RPT)], acc.at[pl.ds(r0, RPT)])
        plsc.subcore_barrier()
        wid = sid * NC + cid
        for j in range(NCH):
            b = wid * BPW + j * CH
            pltpu.sync_copy(idx_hbm.at[pl.ds(b, CH)], idx_v)
            pltpu.sync_copy(msg_hbm.at[pl.ds(b, CH)], msg_v)
            pltpu.sync_copy(msg_v, acc.at[idx_v], add=True)
        plsc.subcore_barrier()
        pltpu.sync_copy(acc.at[pl.ds(r0, RPT)], out_hbm.at[cid, pl.ds(r0, RPT)])

    return _scatter


def _lin0_call(xp, w0t, b0):
    def k(x_ref, w_ref, b_ref, o_ref):
        o_ref[:] = jnp.maximum(
            jnp.dot(x_ref[:], w_ref[:], preferred_element_type=jnp.float32)
            + b_ref[:], 0.0)

    return pl.pallas_call(
        k, out_shape=jax.ShapeDtypeStruct((N, D), jnp.float32))(xp, w0t, b0)


_ET = 8000

def _hid_call(eap, w1t, b1):
    def k(a_ref, w_ref, b_ref, o_ref):
        o_ref[:] = jnp.maximum(
            jnp.dot(a_ref[:], w_ref[:], preferred_element_type=jnp.float32)
            + b_ref[:], 0.0)

    return pl.pallas_call(
        k,
        grid=(E // _ET,),
        in_specs=[
            pl.BlockSpec((_ET, 8), lambda i: (i, 0)),
            pl.BlockSpec((8, 128), lambda i: (0, 0)),
            pl.BlockSpec((1, 128), lambda i: (0, 0)),
        ],
        out_specs=pl.BlockSpec((_ET, 128), lambda i: (i, 0)),
        out_shape=jax.ShapeDtypeStruct((E, 128), jnp.float32),
    )(eap, w1t, b1)


_MT = 2000

def _msg_call(xj, hid, n2t, n2b, first):
    w = 48 if first else D

    def k(xj_ref, h_ref, w_ref, b_ref, o_ref):
        we = jnp.dot(h_ref[:], w_ref[:], preferred_element_type=jnp.float32) + b_ref[:]
        xjv = xj_ref[:]
        acc = we[:, :D] * xjv[:, 0:1]
        for i in range(1, D):
            acc = acc + we[:, i * D:(i + 1) * D] * xjv[:, i:i + 1]
        if first:
            o_ref[:, :D] = acc
            o_ref[:, D:] = jnp.ones((_MT, 16), jnp.float32)
        else:
            o_ref[:] = acc

    return pl.pallas_call(
        k,
        grid=(E // _MT,),
        in_specs=[
            pl.BlockSpec((_MT, D), lambda i: (i, 0)),
            pl.BlockSpec((_MT, 128), lambda i: (i, 0)),
            pl.BlockSpec((128, 1024), lambda i: (0, 0)),
            pl.BlockSpec((1, 1024), lambda i: (0, 0)),
        ],
        out_specs=pl.BlockSpec((_MT, w), lambda i: (i, 0)),
        out_shape=jax.ShapeDtypeStruct((E, w), jnp.float32),
    )(xj, hid, n2t, n2b)


def _gru(out, m, wi_ref, wh_ref, bi_ref, bh_ref):
    gi = jnp.dot(m, wi_ref[:], preferred_element_type=jnp.float32) + bi_ref[:]
    gh = jnp.dot(out, wh_ref[:], preferred_element_type=jnp.float32) + bh_ref[:]
    r = jax.nn.sigmoid(gi[:, :D] + gh[:, :D])
    z = jax.nn.sigmoid(gi[:, D:2 * D] + gh[:, D:2 * D])
    n = jnp.tanh(gi[:, 2 * D:] + r * gh[:, 2 * D:])
    return (1.0 - z) * n + z * out


def _update0_call(part48, out_prev, cw, cb, wih_t, whh_t, bih, bhh):
    def k(p_ref, o_ref, cw_ref, cb_ref, wi_ref, wh_ref, bi_ref, bh_ref,
          h_out, di_out):
        s = p_ref[0] + p_ref[1]
        di = 1.0 / jnp.maximum(s[:, D:D + 1], 1.0)
        aggr = s[:, :D] * di
        out = o_ref[:]
        m = jnp.maximum(
            aggr + jnp.dot(out, cw_ref[:], preferred_element_type=jnp.float32)
            + cb_ref[:], 0.0)
        h_out[:] = _gru(out, m, wi_ref, wh_ref, bi_ref, bh_ref)
        di_out[:] = jnp.broadcast_to(di, (N, D))

    return pl.pallas_call(
        k,
        out_shape=(jax.ShapeDtypeStruct((N, D), jnp.float32),
                   jax.ShapeDtypeStruct((N, D), jnp.float32)),
    )(part48, out_prev, cw, cb, wih_t, whh_t, bih, bhh)


def _update_call(part32, out_prev, di, cw, cb, wih_t, whh_t, bih, bhh):
    def k(p_ref, o_ref, di_ref, cw_ref, cb_ref, wi_ref, wh_ref, bi_ref, bh_ref,
          h_out):
        aggr = (p_ref[0] + p_ref[1]) * di_ref[:]
        out = o_ref[:]
        m = jnp.maximum(
            aggr + jnp.dot(out, cw_ref[:], preferred_element_type=jnp.float32)
            + cb_ref[:], 0.0)
        h_out[:] = _gru(out, m, wi_ref, wh_ref, bi_ref, bh_ref)

    return pl.pallas_call(
        k,
        out_shape=jax.ShapeDtypeStruct((N, D), jnp.float32),
    )(part32, out_prev, di, cw, cb, wih_t, whh_t, bih, bhh)


def _s2s_call(out, batch2, wih_t, whh_t, bih, bhh, l1t, l1b):
    def k(o_ref, b_ref, wi_ref, wh_ref, bi_ref, bh_ref, l1_ref, l1b_ref,
          out_ref):
        out = o_ref[:]
        gidx = lax.broadcasted_iota(jnp.int32, (N, G), 1)
        mf = (b_ref[:] == gidx).astype(jnp.float32)
        q = jnp.zeros((G, 2 * D), jnp.float32)
        hs = jnp.zeros((G, D), jnp.float32)
        cs = jnp.zeros((G, D), jnp.float32)
        for _ in range(3):
            gates = (jnp.dot(q, wi_ref[:], preferred_element_type=jnp.float32)
                     + bi_ref[:]
                     + jnp.dot(hs, wh_ref[:], preferred_element_type=jnp.float32)
                     + bh_ref[:])
            ii = jax.nn.sigmoid(gates[:, :D])
            ff = jax.nn.sigmoid(gates[:, D:2 * D])
            gg = jnp.tanh(gates[:, 2 * D:3 * D])
            oo = jax.nn.sigmoid(gates[:, 3 * D:])
            cs = ff * cs + ii * gg
            hs = oo * jnp.tanh(cs)
            s = lax.dot_general(out, hs, (((1,), (1,)), ((), ())),
                                preferred_element_type=jnp.float32)
            e = jnp.sum(s * mf, axis=1, keepdims=True)
            masked = jnp.where(mf > 0.0, jnp.broadcast_to(e, (N, G)), -1e30)
            emax = jnp.max(masked, axis=0, keepdims=True)
            a = jnp.exp(e - jnp.sum(mf * emax, axis=1, keepdims=True))
            asum = jnp.sum(mf * a, axis=0, keepdims=True)
            an = a / (jnp.sum(mf * asum, axis=1, keepdims=True) + 1e-16)
            r = lax.dot_general(mf * an, out, (((0,), (0,)), ((), ())),
                                preferred_element_type=jnp.float32)
            q = jnp.concatenate([hs, r], axis=1)
        out_ref[:] = jnp.maximum(
            jnp.dot(q, l1_ref[:], preferred_element_type=jnp.float32)
            + l1b_ref[:], 0.0)

    return pl.pallas_call(
        k, out_shape=jax.ShapeDtypeStruct((G, D), jnp.float32),
    )(out, batch2, wih_t, whh_t, bih, bhh, l1t, l1b)


def kernel(x, edge_attr, lin0_W, lin0_b, nn1_W, nn1_b, nn2_W, nn2_b,
           conv_root, conv_b, gru_Wih, gru_Whh, gru_bih, gru_bhh,
           lstm_Wih, lstm_Whh, lstm_bih, lstm_bhh, lin1_W, lin1_b,
           edge_index, batch):
    src = edge_index[0]
    dst = edge_index[1]
    xp = jnp.pad(x, ((0, 0), (0, 5)))
    w0t = jnp.pad(lin0_W, ((0, 0), (0, 5))).T
    b0 = lin0_b.reshape(1, D)
    eap = jnp.pad(edge_attr, ((0, 0), (0, 3)))
    w1t = jnp.pad(nn1_W, ((0, 0), (0, 3))).T
    b1 = nn1_b.reshape(1, 128)
    n2t = nn2_W.T
    n2b = nn2_b.reshape(1, 1024)
    cb = conv_b.reshape(1, D)
    gwih_t = gru_Wih.T
    gwhh_t = gru_Whh.T
    gbih = gru_bih.reshape(1, 3 * D)
    gbhh = gru_bhh.reshape(1, 3 * D)
    lwih_t = lstm_Wih.T
    lwhh_t = lstm_Whh.T
    lbih = lstm_bih.reshape(1, 4 * D)
    lbhh = lstm_bhh.reshape(1, 4 * D)
    l1t = lin1_W.T
    l1b = lin1_b.reshape(1, D)
    zero48 = jnp.zeros((N, 48), jnp.float32)
    zero32 = jnp.zeros((N, D), jnp.float32)

    out = _lin0_call(xp, w0t, b0)
    hid = _hid_call(eap, w1t, b1)

    di = None
    for it in range(3):
        xj = _gather_kernel()(out, src)
        msg = _msg_call(xj, hid, n2t, n2b, first=(it == 0))
        if it == 0:
            part = _scatter_kernel(48)(msg, dst, zero48)
            out, di = _update0_call(part, out, conv_root, cb,
                                    gwih_t, gwhh_t, gbih, gbhh)
        else:
            part = _scatter_kernel(D)(msg, dst, zero32)
            out = _update_call(part, out, di, conv_root, cb,
                               gwih_t, gwhh_t, gbih, gbhh)

    return _s2s_call(out, batch.reshape(N, 1), lwih_t, lwhh_t, lbih, lbhh,
                     l1t, l1b)
